# baseline (speedup 1.0000x reference)
import functools

import jax
import jax.numpy as jnp
from jax import lax
from jax.experimental import pallas as pl
from jax.experimental.pallas import tpu as pltpu
from jax.experimental.pallas import tpu_sc as plsc

DE = 64
TOKEN_SIZE = 100000
BATCH = 4
SEQ = 2048

_info = plsc.get_sparse_core_info()
NC, NS = _info.num_cores, _info.num_subcores
NW = NC * NS
NPASS = DE // NW
GRP = SEQ // 16
HALF = 50048
REM = TOKEN_SIZE - HALF
REMA = 49920
NUNIT = NPASS * 2


def _embed_body(idx_hbm, tok_hbm, pos_hbm, out_hbm,
                idx_v, rh0, rh1, pos_v, out0, out1, sem0, sem1, osem):
    wid = lax.axis_index("s") * NC + lax.axis_index("c")
    dims = [p * NW + wid for p in range(NPASS)]

    rhs = (rh0, rh1)
    sems = (sem0, sem1)
    outs = (out0, out1)

    def fire(u):
        p, h = u // 2, u % 2
        row = tok_hbm.at[dims[p]]
        buf, sem = rhs[u % 2], sems[u % 2]
        if h == 0:
            return [pltpu.async_copy(row.at[pl.ds(0, HALF)], buf, sem)]
        return [pltpu.async_copy(row.at[pl.ds(HALF, REM)], buf, sem)]

    cps = [fire(0)]
    pltpu.sync_copy(idx_hbm, idx_v)
    for p in range(NPASS):
        pltpu.sync_copy(pos_hbm.at[dims[p]], pos_v.at[p])

    out_cps = []
    for u in range(NUNIT):
        p, h = u // 2, u % 2
        for cp in cps[u]:
            cp.wait()
        if u + 1 < NUNIT:
            cps.append(fire(u + 1))
        rh = rhs[u % 2]
        out_v = outs[p]
        lo = h * HALF

        if h == 0:
            def first_half(g, carry, rh=rh, out_v=out_v, p=p):
                sl = pl.ds(g * 16, 16)
                pv = pos_v[p, sl]
                for b in range(BATCH):
                    ids = idx_v[pl.ds(b * SEQ + g * 16, 16)]
                    m = ids < HALF
                    gv = plsc.load_gather(rh, [ids], mask=m)
                    out_v[b, sl] = gv + pv
                return carry

            lax.fori_loop(0, GRP, first_half, 0)
        else:
            def second_half(g, carry, rh=rh, out_v=out_v, p=p, lo=lo):
                sl = pl.ds(g * 16, 16)
                pv = pos_v[p, sl]
                for b in range(BATCH):
                    ids = idx_v[pl.ds(b * SEQ + g * 16, 16)]
                    m = ids >= lo
                    gv = plsc.load_gather(rh, [ids - lo], mask=m)
                    out_v[b, sl] = jnp.where(m, gv + pv, out_v[b, sl])
                return carry

            lax.fori_loop(0, GRP, second_half, 0)

        if h == 1:
            for b in range(BATCH):
                out_cps.append(
                    pltpu.async_copy(out_v.at[b], out_hbm.at[b, dims[p]], osem))

    for cp in out_cps:
        cp.wait()


@functools.partial(
    pl.kernel,
    mesh=plsc.VectorSubcoreMesh(core_axis_name="c", subcore_axis_name="s"),
    out_type=jax.ShapeDtypeStruct((BATCH, DE, SEQ), jnp.float32),
    compiler_params=pltpu.CompilerParams(needs_layout_passes=False),
    scratch_types=[
        pltpu.VMEM((BATCH * SEQ,), jnp.int32),
        pltpu.VMEM((HALF,), jnp.float32),
        pltpu.VMEM((REM,), jnp.float32),
        pltpu.VMEM((NPASS, SEQ), jnp.float32),
        pltpu.VMEM((BATCH, SEQ), jnp.float32),
        pltpu.VMEM((BATCH, SEQ), jnp.float32),
        pltpu.SemaphoreType.DMA,
        pltpu.SemaphoreType.DMA,
        pltpu.SemaphoreType.DMA,
    ],
)
def _embed(idx_hbm, tok_hbm, pos_hbm, out_hbm,
           idx_v, rh0, rh1, pos_v, out0, out1, sem0, sem1, osem):
    _embed_body(idx_hbm, tok_hbm, pos_hbm, out_hbm,
                idx_v, rh0, rh1, pos_v, out0, out1, sem0, sem1, osem)


def kernel(inputs, token_table, pos_table):
    idx = inputs.astype(jnp.int32).reshape(BATCH * SEQ)
    out = _embed(idx, token_table.T, pos_table.T)
    return jnp.transpose(out, (0, 2, 1))


# speedup vs baseline: 1.4376x; 1.4376x over previous
import functools

import jax
import jax.numpy as jnp
from jax import lax
from jax.experimental import pallas as pl
from jax.experimental.pallas import tpu as pltpu
from jax.experimental.pallas import tpu_sc as plsc

DE = 64
TOKEN_SIZE = 100000
BATCH = 4
SEQ = 2048

_info = plsc.get_sparse_core_info()
NC, NS = _info.num_cores, _info.num_subcores
NW = NC * NS
NPASS = DE // NW
GRP = SEQ // 16


def _embed_body(idx_hbm, tok_hbm, pos_hbm, out_hbm,
                idx_v, row_v, pos_v, out0, out1, sem, osem):
    wid = lax.axis_index("s") * NC + lax.axis_index("c")
    dims = [p * NW + wid for p in range(NPASS)]
    outs = (out0, out1)

    row_cp = pltpu.async_copy(tok_hbm.at[dims[0]], row_v, sem)
    pltpu.sync_copy(idx_hbm, idx_v)
    for p in range(NPASS):
        pltpu.sync_copy(pos_hbm.at[dims[p]], pos_v.at[p])

    out_cps = []
    for p in range(NPASS):
        row_cp.wait()
        out_v = outs[p]

        @plsc.parallel_loop(0, GRP, unroll=4)
        def gather_add(g, out_v=out_v, p=p):
            sl = pl.ds(g * 16, 16)
            pv = pos_v[p, sl]
            for b in range(BATCH):
                ids = idx_v[pl.ds(b * SEQ + g * 16, 16)]
                out_v[b, sl] = plsc.load_gather(row_v, [ids]) + pv

        if p + 1 < NPASS:
            row_cp = pltpu.async_copy(tok_hbm.at[dims[p + 1]], row_v, sem)
        for b in range(BATCH):
            out_cps.append(
                pltpu.async_copy(out_v.at[b], out_hbm.at[b, dims[p]], osem))

    for cp in out_cps:
        cp.wait()


@functools.partial(
    pl.kernel,
    mesh=plsc.VectorSubcoreMesh(core_axis_name="c", subcore_axis_name="s"),
    out_type=jax.ShapeDtypeStruct((BATCH, DE, SEQ), jnp.float32),
    compiler_params=pltpu.CompilerParams(needs_layout_passes=False),
    scratch_types=[
        pltpu.VMEM((BATCH * SEQ,), jnp.int32),
        pltpu.VMEM((TOKEN_SIZE,), jnp.float32),
        pltpu.VMEM((NPASS, SEQ), jnp.float32),
        pltpu.VMEM((BATCH, SEQ), jnp.float32),
        pltpu.VMEM((BATCH, SEQ), jnp.float32),
        pltpu.SemaphoreType.DMA,
        pltpu.SemaphoreType.DMA,
    ],
)
def _embed(idx_hbm, tok_hbm, pos_hbm, out_hbm,
           idx_v, row_v, pos_v, out0, out1, sem, osem):
    _embed_body(idx_hbm, tok_hbm, pos_hbm, out_hbm,
                idx_v, row_v, pos_v, out0, out1, sem, osem)


def kernel(inputs, token_table, pos_table):
    idx = inputs.astype(jnp.int32).reshape(BATCH * SEQ)
    out = _embed(idx, token_table.T, pos_table.T)
    return jnp.transpose(out, (0, 2, 1))


# speedup vs baseline: 1.4392x; 1.0011x over previous
import functools

import jax
import jax.numpy as jnp
from jax import lax
from jax.experimental import pallas as pl
from jax.experimental.pallas import tpu as pltpu
from jax.experimental.pallas import tpu_sc as plsc

DE = 64
TOKEN_SIZE = 100000
BATCH = 4
SEQ = 2048

_info = plsc.get_sparse_core_info()
NC, NS = _info.num_cores, _info.num_subcores
NW = NC * NS
NPASS = DE // NW
GRP = SEQ // 16


def _embed_body(idx_hbm, tok_hbm, pos_hbm, out_hbm,
                idx_v, row_v, pos_v, out0, out1, sem, osem):
    wid = lax.axis_index("s") * NC + lax.axis_index("c")
    dims = [p * NW + wid for p in range(NPASS)]
    outs = (out0, out1)

    row_cp = pltpu.async_copy(tok_hbm.at[dims[0]], row_v, sem)
    pltpu.sync_copy(idx_hbm, idx_v)
    for p in range(NPASS):
        pltpu.sync_copy(pos_hbm.at[dims[p]], pos_v.at[p])

    out_cps = []
    for p in range(NPASS):
        row_cp.wait()
        out_v = outs[p]

        @plsc.parallel_loop(0, GRP, unroll=4)
        def gather_add(g, out_v=out_v, p=p):
            sl = pl.ds(g * 16, 16)
            csl = pl.ds((g % 8) * 16, 16)
            pv = pos_v[p, sl]
            for b in range(BATCH):
                ids = idx_v[(g // 8) * BATCH + b, csl]
                out_v[b, sl] = plsc.load_gather(row_v, [ids]) + pv

        if p + 1 < NPASS:
            row_cp = pltpu.async_copy(tok_hbm.at[dims[p + 1]], row_v, sem)
        for b in range(BATCH):
            out_cps.append(
                pltpu.async_copy(out_v.at[b], out_hbm.at[b, dims[p]], osem))

    for cp in out_cps:
        cp.wait()


@functools.partial(
    pl.kernel,
    mesh=plsc.VectorSubcoreMesh(core_axis_name="c", subcore_axis_name="s"),
    out_type=jax.ShapeDtypeStruct((BATCH, DE, SEQ), jnp.float32),
    compiler_params=pltpu.CompilerParams(needs_layout_passes=False),
    scratch_types=[
        pltpu.VMEM((BATCH * SEQ // 128, 128), jnp.int32),
        pltpu.VMEM((TOKEN_SIZE,), jnp.float32),
        pltpu.VMEM((NPASS, SEQ), jnp.float32),
        pltpu.VMEM((BATCH, SEQ), jnp.float32),
        pltpu.VMEM((BATCH, SEQ), jnp.float32),
        pltpu.SemaphoreType.DMA,
        pltpu.SemaphoreType.DMA,
    ],
)
def _embed(idx_hbm, tok_hbm, pos_hbm, out_hbm,
           idx_v, row_v, pos_v, out0, out1, sem, osem):
    _embed_body(idx_hbm, tok_hbm, pos_hbm, out_hbm,
                idx_v, row_v, pos_v, out0, out1, sem, osem)


def kernel(inputs, token_table, pos_table):
    idx = inputs.astype(jnp.int32).reshape(BATCH, SEQ // 128, 128)
    idx = jnp.transpose(idx, (1, 0, 2)).reshape(BATCH * SEQ // 128, 128)
    out = _embed(idx, token_table.T, pos_table.T)
    return jnp.transpose(out, (0, 2, 1))
